# baseline (speedup 1.0000x reference)
import functools

import jax
import jax.numpy as jnp
from jax import lax
from jax.experimental import pallas as pl
from jax.experimental.pallas import tpu as pltpu
from jax.experimental.pallas import tpu_sc as plsc

D = H = W = 64
PD = D + 2
NP = PD * PD * PD
GR = NP + 8
DUMP = NP
CIN = 64
CH = 32
CROW = 128
COUT = 4 * CH
PLANE = PD * PD

NW = 32
NPAD = 50176
PER_TILE = NPAD // NW
KCH = 784

BZ = 4

_sc_mesh = plsc.VectorSubcoreMesh(core_axis_name="c", subcore_axis_name="s")


@functools.partial(
    pl.kernel,
    out_type=jax.ShapeDtypeStruct((GR, CROW), jnp.float32),
    mesh=_sc_mesh,
    scratch_types=[
        pltpu.VMEM((KCH,), jnp.int32),
        pltpu.VMEM((KCH, CROW), jnp.float32),
        pltpu.SemaphoreType.DMA,
    ],
)
def _sc_scatter(idx_hbm, payload_hbm, grid_hbm, idx_v, rows_v, sem):
    wid = lax.axis_index("s") * 2 + lax.axis_index("c")
    for j in range(PER_TILE // KCH):
        base = wid * PER_TILE + j * KCH
        pltpu.sync_copy(idx_hbm.at[pl.ds(base, KCH)], idx_v)
        pltpu.sync_copy(payload_hbm.at[pl.ds(base, KCH)], rows_v)
        pltpu.async_copy(rows_v, grid_hbm.at[idx_v], sem).wait()


@functools.partial(
    pl.kernel,
    out_type=jax.ShapeDtypeStruct((NPAD, COUT), jnp.float32),
    mesh=_sc_mesh,
    scratch_types=[
        pltpu.VMEM((KCH,), jnp.int32),
        pltpu.VMEM((KCH, COUT), jnp.float32),
        pltpu.SemaphoreType.DMA,
    ],
)
def _sc_gather(idx_hbm, conv_hbm, out_hbm, idx_v, rows_v, sem):
    wid = lax.axis_index("s") * 2 + lax.axis_index("c")
    for j in range(PER_TILE // KCH):
        base = wid * PER_TILE + j * KCH
        pltpu.sync_copy(idx_hbm.at[pl.ds(base, KCH)], idx_v)
        pltpu.async_copy(conv_hbm.at[idx_v], rows_v, sem).wait()
        pltpu.sync_copy(rows_v, out_hbm.at[pl.ds(base, KCH)])


def _conv_body(pa, pb, ma, mb, w_ref, b_ref, out_ref, sm):
    for p in range(BZ + 2):
        if p < BZ:
            src = pa[p * PLANE:(p + 1) * PLANE, :]
            m = ma[p, 0]
        else:
            src = pb[(p - BZ) * PLANE:(p - BZ + 1) * PLANE, :]
            m = mb[p - BZ, 0]
        keep = m.reshape(PLANE, 1) > 0.0
        sm[p] = jnp.where(keep, src, 0.0).astype(jnp.bfloat16).reshape(
            PD, PD, CROW)
    for oz in range(BZ):
        acc = jnp.broadcast_to(b_ref[0], (D * W, COUT)).astype(jnp.float32)
        for dzi in range(3):
            for dyi in range(3):
                for dxi in range(3):
                    ko = (dzi * 3 + dyi) * 3 + dxi
                    a = sm[oz + dzi, dyi:dyi + D, dxi:dxi + W, :].reshape(
                        D * W, CROW)
                    acc = acc + jnp.dot(a, w_ref[ko],
                                        preferred_element_type=jnp.float32)
        out_ref[oz * D * W:(oz + 1) * D * W, :] = acc


def _conv(grid, mask, weight_p, bias):
    return pl.pallas_call(
        _conv_body,
        grid=(D // BZ,),
        in_specs=[
            pl.BlockSpec((BZ * PLANE, CROW), lambda z: (z, 0)),
            pl.BlockSpec((2 * PLANE, CROW), lambda z: ((BZ // 2) * z + 2, 0)),
            pl.BlockSpec((BZ, 1, PLANE), lambda z: (z, 0, 0)),
            pl.BlockSpec((2, 1, PLANE), lambda z: ((BZ // 2) * z + 2, 0, 0)),
            pl.BlockSpec((27, CROW, COUT), lambda z: (0, 0, 0)),
            pl.BlockSpec((1, COUT), lambda z: (0, 0)),
        ],
        out_specs=pl.BlockSpec((BZ * D * W, COUT), lambda z: (z, 0)),
        out_shape=jax.ShapeDtypeStruct((D * H * W, COUT), jnp.float32),
        scratch_shapes=[pltpu.VMEM((BZ + 2, PD, PD, CROW), jnp.bfloat16)],
    )(grid, grid, mask, mask, weight_p, bias.reshape(1, COUT))


def _gates0_body(rows_ref, fnext_ref, h_ref, c_ref, pay_ref):
    rows = rows_ref[...]
    i = jax.nn.sigmoid(rows[:, 0 * CH:1 * CH])
    f = jax.nn.sigmoid(rows[:, 1 * CH:2 * CH])
    o = jax.nn.sigmoid(rows[:, 2 * CH:3 * CH])
    g = jnp.tanh(rows[:, 3 * CH:4 * CH])
    c = i * g
    del f
    h = o * jnp.tanh(c)
    h_ref[...] = h
    c_ref[...] = c
    pay_ref[:, 0:CIN] = fnext_ref[...]
    pay_ref[:, CIN:CIN + CH] = h
    pay_ref[:, CIN + CH:CROW] = jnp.zeros_like(h)


def _gates1_body(rows_ref, cprev_ref, h_ref, c_ref):
    rows = rows_ref[...]
    i = jax.nn.sigmoid(rows[:, 0 * CH:1 * CH])
    f = jax.nn.sigmoid(rows[:, 1 * CH:2 * CH])
    o = jax.nn.sigmoid(rows[:, 2 * CH:3 * CH])
    g = jnp.tanh(rows[:, 3 * CH:4 * CH])
    c = f * cprev_ref[...] + i * g
    h_ref[...] = h = o * jnp.tanh(c)
    c_ref[...] = c


_GBLK = 3136


def _gates0(rows, fnext):
    return pl.pallas_call(
        _gates0_body,
        grid=(NPAD // _GBLK,),
        in_specs=[
            pl.BlockSpec((_GBLK, COUT), lambda i: (i, 0)),
            pl.BlockSpec((_GBLK, CIN), lambda i: (i, 0)),
        ],
        out_specs=[
            pl.BlockSpec((_GBLK, CH), lambda i: (i, 0)),
            pl.BlockSpec((_GBLK, CH), lambda i: (i, 0)),
            pl.BlockSpec((_GBLK, CROW), lambda i: (i, 0)),
        ],
        out_shape=[
            jax.ShapeDtypeStruct((NPAD, CH), jnp.float32),
            jax.ShapeDtypeStruct((NPAD, CH), jnp.float32),
            jax.ShapeDtypeStruct((NPAD, CROW), jnp.float32),
        ],
    )(rows, fnext)


def _gates1(rows, cprev):
    return pl.pallas_call(
        _gates1_body,
        grid=(NPAD // _GBLK,),
        in_specs=[
            pl.BlockSpec((_GBLK, COUT), lambda i: (i, 0)),
            pl.BlockSpec((_GBLK, CH), lambda i: (i, 0)),
        ],
        out_specs=[
            pl.BlockSpec((_GBLK, CH), lambda i: (i, 0)),
            pl.BlockSpec((_GBLK, CH), lambda i: (i, 0)),
        ],
        out_shape=[
            jax.ShapeDtypeStruct((NPAD, CH), jnp.float32),
            jax.ShapeDtypeStruct((NPAD, CH), jnp.float32),
        ],
    )(rows, cprev)


def kernel(features, coords, weight, bias):
    N = features.shape[1]
    z, y, x = coords[:, 0], coords[:, 1], coords[:, 2]
    lin64 = (z * H + y) * W + x
    plin = ((z + 1) * PD + (y + 1)) * PD + (x + 1)

    idxmap = jnp.full((D * H * W,), -1, jnp.int32).at[lin64].set(
        jnp.arange(N, dtype=jnp.int32))
    winner = idxmap[lin64] == jnp.arange(N, dtype=jnp.int32)
    plin_eff = jnp.where(winner, plin, DUMP)
    plin_pad = jnp.full((NPAD,), DUMP, jnp.int32).at[:N].set(plin_eff)
    lin_pad = jnp.zeros((NPAD,), jnp.int32).at[:N].set(lin64)

    mask = jnp.zeros((NP,), jnp.float32).at[plin].set(1.0).reshape(
        PD, 1, PLANE)

    weight_p = jnp.zeros((27, CROW, COUT), jnp.bfloat16)
    weight_p = weight_p.at[:, :CIN + CH, :].set(weight.astype(jnp.bfloat16))

    fpad = jnp.zeros((2, NPAD, CIN), jnp.float32).at[:, :N, :].set(features)
    pay0 = jnp.pad(fpad[0], ((0, 0), (0, CROW - CIN)))

    grid0 = _sc_scatter(plin_pad, pay0)
    conv0 = _conv(grid0, mask, weight_p, bias)
    rows0 = _sc_gather(lin_pad, conv0)
    h0, c0, pay1 = _gates0(rows0, fpad[1])

    grid1 = _sc_scatter(plin_pad, pay1)
    conv1 = _conv(grid1, mask, weight_p, bias)
    rows1 = _sc_gather(lin_pad, conv1)
    h1, c1 = _gates1(rows1, c0)

    outs = jnp.stack([h0[:N], h1[:N]])
    return (outs, h1[:N], c1[:N])


# speedup vs baseline: 1.0683x; 1.0683x over previous
import functools

import jax
import jax.numpy as jnp
from jax import lax
from jax.experimental import pallas as pl
from jax.experimental.pallas import tpu as pltpu
from jax.experimental.pallas import tpu_sc as plsc

D = H = W = 64
PD = D + 2
NP = PD * PD * PD
CIN = 64
CH = 32
CROW = 128
COUT = 4 * CH
PLANE = PD * PD
PLANE_S = 4368

NW = 32
NPAD = 50176
PER_TILE = NPAD // NW
KCH = 784
NPS = PD * PLANE_S
GR = NPS + NPAD

BZ = 4

_sc_mesh = plsc.VectorSubcoreMesh(core_axis_name="c", subcore_axis_name="s")


@functools.partial(
    pl.kernel,
    out_type=jax.ShapeDtypeStruct((GR, CROW), jnp.float32),
    mesh=_sc_mesh,
    scratch_types=[
        pltpu.VMEM((KCH,), jnp.int32),
        pltpu.VMEM((KCH, CROW), jnp.float32),
        pltpu.SemaphoreType.DMA,
    ],
)
def _sc_scatter(idx_hbm, payload_hbm, grid_hbm, idx_v, rows_v, sem):
    wid = lax.axis_index("s") * 2 + lax.axis_index("c")
    for j in range(PER_TILE // KCH):
        base = wid * PER_TILE + j * KCH
        pltpu.sync_copy(idx_hbm.at[pl.ds(base, KCH)], idx_v)
        pltpu.sync_copy(payload_hbm.at[pl.ds(base, KCH)], rows_v)
        pltpu.async_copy(rows_v, grid_hbm.at[idx_v], sem).wait()


@functools.partial(
    pl.kernel,
    out_type=jax.ShapeDtypeStruct((NPAD, COUT), jnp.float32),
    mesh=_sc_mesh,
    scratch_types=[
        pltpu.VMEM((KCH,), jnp.int32),
        pltpu.VMEM((KCH, COUT), jnp.float32),
        pltpu.SemaphoreType.DMA,
    ],
)
def _sc_gather(idx_hbm, conv_hbm, out_hbm, idx_v, rows_v, sem):
    wid = lax.axis_index("s") * 2 + lax.axis_index("c")
    for j in range(PER_TILE // KCH):
        base = wid * PER_TILE + j * KCH
        pltpu.sync_copy(idx_hbm.at[pl.ds(base, KCH)], idx_v)
        pltpu.async_copy(conv_hbm.at[idx_v], rows_v, sem).wait()
        pltpu.sync_copy(rows_v, out_hbm.at[pl.ds(base, KCH)])


def _conv_body(pa, pb, ma, mb, w_ref, b_ref, out_ref, sm):
    for p in range(BZ + 2):
        if p < BZ:
            src = pa[p * PLANE_S:p * PLANE_S + PLANE, :]
            m = ma[p, 0, :PLANE]
        else:
            src = pb[(p - BZ) * PLANE_S:(p - BZ) * PLANE_S + PLANE, :]
            m = mb[p - BZ, 0, :PLANE]
        keep = m.reshape(PLANE, 1) > 0.0
        sm[p] = jnp.where(keep, src, 0.0).astype(jnp.bfloat16).reshape(
            PD, PD, CROW)
    for oz in range(BZ):
        acc = jnp.broadcast_to(b_ref[0], (D * W, COUT)).astype(jnp.float32)
        for dzi in range(3):
            for dyi in range(3):
                for dxi in range(3):
                    ko = (dzi * 3 + dyi) * 3 + dxi
                    a = sm[oz + dzi, dyi:dyi + D, dxi:dxi + W, :].reshape(
                        D * W, CROW)
                    acc = acc + jnp.dot(a, w_ref[ko],
                                        preferred_element_type=jnp.float32)
        out_ref[oz * D * W:(oz + 1) * D * W, :] = acc


def _conv(grid, mask, weight_p, bias):
    return pl.pallas_call(
        _conv_body,
        grid=(D // BZ,),
        in_specs=[
            pl.BlockSpec((BZ * PLANE_S, CROW), lambda z: (z, 0)),
            pl.BlockSpec((2 * PLANE_S, CROW),
                         lambda z: ((BZ // 2) * z + 2, 0)),
            pl.BlockSpec((BZ, 1, PLANE_S), lambda z: (z, 0, 0)),
            pl.BlockSpec((2, 1, PLANE_S), lambda z: ((BZ // 2) * z + 2, 0, 0)),
            pl.BlockSpec((27, CROW, COUT), lambda z: (0, 0, 0)),
            pl.BlockSpec((1, COUT), lambda z: (0, 0)),
        ],
        out_specs=pl.BlockSpec((BZ * D * W, COUT), lambda z: (z, 0)),
        out_shape=jax.ShapeDtypeStruct((D * H * W, COUT), jnp.float32),
        scratch_shapes=[pltpu.VMEM((BZ + 2, PD, PD, CROW), jnp.bfloat16)],
    )(grid, grid, mask, mask, weight_p, bias.reshape(1, COUT))


def _gates0_body(rows_ref, fnext_ref, h_ref, c_ref, pay_ref):
    rows = rows_ref[...]
    i = jax.nn.sigmoid(rows[:, 0 * CH:1 * CH])
    f = jax.nn.sigmoid(rows[:, 1 * CH:2 * CH])
    o = jax.nn.sigmoid(rows[:, 2 * CH:3 * CH])
    g = jnp.tanh(rows[:, 3 * CH:4 * CH])
    c = i * g
    del f
    h = o * jnp.tanh(c)
    h_ref[...] = h
    c_ref[...] = c
    pay_ref[:, 0:CIN] = fnext_ref[...]
    pay_ref[:, CIN:CIN + CH] = h
    pay_ref[:, CIN + CH:CROW] = jnp.zeros((_GBLK, CH), jnp.float32)


def _gates1_body(rows_ref, cprev_ref, h_ref, c_ref):
    rows = rows_ref[...]
    i = jax.nn.sigmoid(rows[:, 0 * CH:1 * CH])
    f = jax.nn.sigmoid(rows[:, 1 * CH:2 * CH])
    o = jax.nn.sigmoid(rows[:, 2 * CH:3 * CH])
    g = jnp.tanh(rows[:, 3 * CH:4 * CH])
    c = f * cprev_ref[...] + i * g
    h_ref[...] = h = o * jnp.tanh(c)
    c_ref[...] = c


_GBLK = 3136


def _gates0(rows, fnext):
    return pl.pallas_call(
        _gates0_body,
        grid=(NPAD // _GBLK,),
        in_specs=[
            pl.BlockSpec((_GBLK, COUT), lambda i: (i, 0)),
            pl.BlockSpec((_GBLK, CIN), lambda i: (i, 0)),
        ],
        out_specs=[
            pl.BlockSpec((_GBLK, CH), lambda i: (i, 0)),
            pl.BlockSpec((_GBLK, CH), lambda i: (i, 0)),
            pl.BlockSpec((_GBLK, CROW), lambda i: (i, 0)),
        ],
        out_shape=[
            jax.ShapeDtypeStruct((NPAD, CH), jnp.float32),
            jax.ShapeDtypeStruct((NPAD, CH), jnp.float32),
            jax.ShapeDtypeStruct((NPAD, CROW), jnp.float32),
        ],
    )(rows, fnext)


def _gates1(rows, cprev):
    return pl.pallas_call(
        _gates1_body,
        grid=(NPAD // _GBLK,),
        in_specs=[
            pl.BlockSpec((_GBLK, COUT), lambda i: (i, 0)),
            pl.BlockSpec((_GBLK, CH), lambda i: (i, 0)),
        ],
        out_specs=[
            pl.BlockSpec((_GBLK, CH), lambda i: (i, 0)),
            pl.BlockSpec((_GBLK, CH), lambda i: (i, 0)),
        ],
        out_shape=[
            jax.ShapeDtypeStruct((NPAD, CH), jnp.float32),
            jax.ShapeDtypeStruct((NPAD, CH), jnp.float32),
        ],
    )(rows, cprev)


def kernel(features, coords, weight, bias):
    N = features.shape[1]
    z, y, x = coords[:, 0], coords[:, 1], coords[:, 2]
    lin64 = (z * H + y) * W + x
    plin = (z + 1) * PLANE_S + (y + 1) * PD + (x + 1)

    idxmap = jnp.full((D * H * W,), -1, jnp.int32).at[lin64].set(
        jnp.arange(N, dtype=jnp.int32))
    winner = idxmap[lin64] == jnp.arange(N, dtype=jnp.int32)
    iarange = jnp.arange(NPAD, dtype=jnp.int32)
    winner_pad = jnp.zeros((NPAD,), jnp.bool_).at[:N].set(winner)
    plin_full = jnp.zeros((NPAD,), jnp.int32).at[:N].set(plin)
    plin_pad = jnp.where(winner_pad, plin_full, NPS + iarange)
    lin_pad = jnp.zeros((NPAD,), jnp.int32).at[:N].set(lin64)

    mask = jnp.zeros((NPS,), jnp.float32).at[plin].set(1.0).reshape(
        PD, 1, PLANE_S)

    weight_p = jnp.zeros((27, CROW, COUT), jnp.bfloat16)
    weight_p = weight_p.at[:, :CIN + CH, :].set(weight.astype(jnp.bfloat16))

    fpad = jnp.zeros((2, NPAD, CIN), jnp.float32).at[:, :N, :].set(features)
    pay0 = jnp.pad(fpad[0], ((0, 0), (0, CROW - CIN)))

    grid0 = _sc_scatter(plin_pad, pay0)
    conv0 = _conv(grid0, mask, weight_p, bias)
    rows0 = _sc_gather(lin_pad, conv0)
    h0, c0, pay1 = _gates0(rows0, fpad[1])

    grid1 = _sc_scatter(plin_pad, pay1)
    conv1 = _conv(grid1, mask, weight_p, bias)
    rows1 = _sc_gather(lin_pad, conv1)
    h1, c1 = _gates1(rows1, c0)

    outs = jnp.stack([h0[:N], h1[:N]])
    return (outs, h1[:N], c1[:N])


# speedup vs baseline: 1.0970x; 1.0268x over previous
import functools

import jax
import jax.numpy as jnp
from jax import lax
from jax.experimental import pallas as pl
from jax.experimental.pallas import tpu as pltpu
from jax.experimental.pallas import tpu_sc as plsc

D = H = W = 64
PD = D + 2
NP = PD * PD * PD
CIN = 64
CH = 32
CROW = 128
COUT = 4 * CH
PLANE = PD * PD
PLANE_S = 4368

NW = 32
NPAD = 50176
PER_TILE = NPAD // NW
KCH = 784
NPS = PD * PLANE_S
GR = NPS + NPAD

BZ = 4

_sc_mesh = plsc.VectorSubcoreMesh(core_axis_name="c", subcore_axis_name="s")


@functools.partial(
    pl.kernel,
    out_type=jax.ShapeDtypeStruct((GR, CROW), jnp.float32),
    mesh=_sc_mesh,
    scratch_types=[
        pltpu.VMEM((KCH,), jnp.int32),
        pltpu.VMEM((KCH, CROW), jnp.float32),
        pltpu.SemaphoreType.DMA,
    ],
)
def _sc_scatter(idx_hbm, payload_hbm, grid_hbm, idx_v, rows_v, sem):
    wid = lax.axis_index("s") * 2 + lax.axis_index("c")
    for j in range(PER_TILE // KCH):
        base = wid * PER_TILE + j * KCH
        pltpu.sync_copy(idx_hbm.at[pl.ds(base, KCH)], idx_v)
        pltpu.sync_copy(payload_hbm.at[pl.ds(base, KCH)], rows_v)
        pltpu.async_copy(rows_v, grid_hbm.at[idx_v], sem).wait()


@functools.partial(
    pl.kernel,
    out_type=jax.ShapeDtypeStruct((NPAD, COUT), jnp.float32),
    mesh=_sc_mesh,
    scratch_types=[
        pltpu.VMEM((KCH,), jnp.int32),
        pltpu.VMEM((KCH, COUT), jnp.float32),
        pltpu.SemaphoreType.DMA,
    ],
)
def _sc_gather(idx_hbm, conv_hbm, out_hbm, idx_v, rows_v, sem):
    wid = lax.axis_index("s") * 2 + lax.axis_index("c")
    for j in range(PER_TILE // KCH):
        base = wid * PER_TILE + j * KCH
        pltpu.sync_copy(idx_hbm.at[pl.ds(base, KCH)], idx_v)
        pltpu.async_copy(conv_hbm.at[idx_v], rows_v, sem).wait()
        pltpu.sync_copy(rows_v, out_hbm.at[pl.ds(base, KCH)])


def _conv_body(pa, pb, ma, mb, w_ref, b_ref, out_ref, sm):
    for p in range(BZ + 2):
        if p < BZ:
            src = pa[p * PLANE_S:p * PLANE_S + PLANE, :]
            m = ma[p, 0, :PLANE]
        else:
            src = pb[(p - BZ) * PLANE_S:(p - BZ) * PLANE_S + PLANE, :]
            m = mb[p - BZ, 0, :PLANE]
        keep = m.reshape(PLANE, 1) > 0.0
        sm[p] = jnp.where(keep, src, 0.0).astype(jnp.bfloat16).reshape(
            PD, PD, CROW)
    for oz in range(BZ):
        acc = jnp.broadcast_to(b_ref[0], (D * W, COUT)).astype(jnp.float32)
        for dzi in range(3):
            for dyi in range(3):
                g = dzi * 3 + dyi
                a = jnp.concatenate(
                    [sm[oz + dzi, dyi:dyi + D, dxi:dxi + W, :].reshape(
                        D * W, CROW) for dxi in range(3)], axis=1)
                acc = acc + jnp.dot(a, w_ref[g],
                                    preferred_element_type=jnp.float32)
        out_ref[oz * D * W:(oz + 1) * D * W, :] = acc


def _conv(grid, mask, weight_p, bias):
    return pl.pallas_call(
        _conv_body,
        grid=(D // BZ,),
        in_specs=[
            pl.BlockSpec((BZ * PLANE_S, CROW), lambda z: (z, 0)),
            pl.BlockSpec((2 * PLANE_S, CROW),
                         lambda z: ((BZ // 2) * z + 2, 0)),
            pl.BlockSpec((BZ, 1, PLANE_S), lambda z: (z, 0, 0)),
            pl.BlockSpec((2, 1, PLANE_S), lambda z: ((BZ // 2) * z + 2, 0, 0)),
            pl.BlockSpec((9, 3 * CROW, COUT), lambda z: (0, 0, 0)),
            pl.BlockSpec((1, COUT), lambda z: (0, 0)),
        ],
        out_specs=pl.BlockSpec((BZ * D * W, COUT), lambda z: (z, 0)),
        out_shape=jax.ShapeDtypeStruct((D * H * W, COUT), jnp.float32),
        scratch_shapes=[pltpu.VMEM((BZ + 2, PD, PD, CROW), jnp.bfloat16)],
    )(grid, grid, mask, mask, weight_p, bias.reshape(1, COUT))


def _gates0_body(rows_ref, fnext_ref, h_ref, c_ref, pay_ref):
    rows = rows_ref[...]
    i = jax.nn.sigmoid(rows[:, 0 * CH:1 * CH])
    f = jax.nn.sigmoid(rows[:, 1 * CH:2 * CH])
    o = jax.nn.sigmoid(rows[:, 2 * CH:3 * CH])
    g = jnp.tanh(rows[:, 3 * CH:4 * CH])
    c = i * g
    del f
    h = o * jnp.tanh(c)
    h_ref[...] = h
    c_ref[...] = c
    pay_ref[:, 0:CIN] = fnext_ref[...]
    pay_ref[:, CIN:CIN + CH] = h
    pay_ref[:, CIN + CH:CROW] = jnp.zeros((_GBLK, CH), jnp.float32)


def _gates1_body(rows_ref, cprev_ref, h_ref, c_ref):
    rows = rows_ref[...]
    i = jax.nn.sigmoid(rows[:, 0 * CH:1 * CH])
    f = jax.nn.sigmoid(rows[:, 1 * CH:2 * CH])
    o = jax.nn.sigmoid(rows[:, 2 * CH:3 * CH])
    g = jnp.tanh(rows[:, 3 * CH:4 * CH])
    c = f * cprev_ref[...] + i * g
    h_ref[...] = h = o * jnp.tanh(c)
    c_ref[...] = c


_GBLK = 3136


def _gates0(rows, fnext):
    return pl.pallas_call(
        _gates0_body,
        grid=(NPAD // _GBLK,),
        in_specs=[
            pl.BlockSpec((_GBLK, COUT), lambda i: (i, 0)),
            pl.BlockSpec((_GBLK, CIN), lambda i: (i, 0)),
        ],
        out_specs=[
            pl.BlockSpec((_GBLK, CH), lambda i: (i, 0)),
            pl.BlockSpec((_GBLK, CH), lambda i: (i, 0)),
            pl.BlockSpec((_GBLK, CROW), lambda i: (i, 0)),
        ],
        out_shape=[
            jax.ShapeDtypeStruct((NPAD, CH), jnp.float32),
            jax.ShapeDtypeStruct((NPAD, CH), jnp.float32),
            jax.ShapeDtypeStruct((NPAD, CROW), jnp.float32),
        ],
    )(rows, fnext)


def _gates1(rows, cprev):
    return pl.pallas_call(
        _gates1_body,
        grid=(NPAD // _GBLK,),
        in_specs=[
            pl.BlockSpec((_GBLK, COUT), lambda i: (i, 0)),
            pl.BlockSpec((_GBLK, CH), lambda i: (i, 0)),
        ],
        out_specs=[
            pl.BlockSpec((_GBLK, CH), lambda i: (i, 0)),
            pl.BlockSpec((_GBLK, CH), lambda i: (i, 0)),
        ],
        out_shape=[
            jax.ShapeDtypeStruct((NPAD, CH), jnp.float32),
            jax.ShapeDtypeStruct((NPAD, CH), jnp.float32),
        ],
    )(rows, cprev)


def kernel(features, coords, weight, bias):
    N = features.shape[1]
    z, y, x = coords[:, 0], coords[:, 1], coords[:, 2]
    lin64 = (z * H + y) * W + x
    plin = (z + 1) * PLANE_S + (y + 1) * PD + (x + 1)

    idxmap = jnp.full((D * H * W,), -1, jnp.int32).at[lin64].set(
        jnp.arange(N, dtype=jnp.int32))
    winner = idxmap[lin64] == jnp.arange(N, dtype=jnp.int32)
    iarange = jnp.arange(NPAD, dtype=jnp.int32)
    winner_pad = jnp.zeros((NPAD,), jnp.bool_).at[:N].set(winner)
    plin_full = jnp.zeros((NPAD,), jnp.int32).at[:N].set(plin)
    plin_pad = jnp.where(winner_pad, plin_full, NPS + iarange)
    lin_pad = jnp.zeros((NPAD,), jnp.int32).at[:N].set(lin64)

    mask = jnp.zeros((NPS,), jnp.float32).at[plin].set(1.0).reshape(
        PD, 1, PLANE_S)

    weight_p = jnp.zeros((27, CROW, COUT), jnp.bfloat16)
    weight_p = weight_p.at[:, :CIN + CH, :].set(weight.astype(jnp.bfloat16))
    weight_p = weight_p.reshape(9, 3 * CROW, COUT)

    fpad = jnp.zeros((2, NPAD, CIN), jnp.float32).at[:, :N, :].set(features)
    pay0 = jnp.pad(fpad[0], ((0, 0), (0, CROW - CIN)))

    grid0 = _sc_scatter(plin_pad, pay0)
    conv0 = _conv(grid0, mask, weight_p, bias)
    rows0 = _sc_gather(lin_pad, conv0)
    h0, c0, pay1 = _gates0(rows0, fpad[1])

    grid1 = _sc_scatter(plin_pad, pay1)
    conv1 = _conv(grid1, mask, weight_p, bias)
    rows1 = _sc_gather(lin_pad, conv1)
    h1, c1 = _gates1(rows1, c0)

    outs = jnp.stack([h0[:N], h1[:N]])
    return (outs, h1[:N], c1[:N])


# speedup vs baseline: 1.3169x; 1.2005x over previous
import functools

import jax
import jax.numpy as jnp
from jax import lax
from jax.experimental import pallas as pl
from jax.experimental.pallas import tpu as pltpu
from jax.experimental.pallas import tpu_sc as plsc

D = H = W = 64
PD = D + 2
NP = PD * PD * PD
CIN = 64
CH = 32
CROW = 128
COUT = 4 * CH
PLANE = PD * PD
PLANE_S = 4368

NW = 32
NPAD = 50176
PER_TILE = NPAD // NW
KCH = 784
NPS = PD * PLANE_S
GR = NPS + NPAD

BZ = 2

_sc_mesh = plsc.VectorSubcoreMesh(core_axis_name="c", subcore_axis_name="s")


@functools.partial(
    pl.kernel,
    out_type=jax.ShapeDtypeStruct((GR, CROW), jnp.float32),
    mesh=_sc_mesh,
    scratch_types=[
        pltpu.VMEM((KCH,), jnp.int32),
        pltpu.VMEM((KCH, CROW), jnp.float32),
        pltpu.SemaphoreType.DMA,
    ],
)
def _sc_scatter(idx_hbm, payload_hbm, grid_hbm, idx_v, rows_v, sem):
    wid = lax.axis_index("s") * 2 + lax.axis_index("c")
    for j in range(PER_TILE // KCH):
        base = wid * PER_TILE + j * KCH
        pltpu.sync_copy(idx_hbm.at[pl.ds(base, KCH)], idx_v)
        pltpu.sync_copy(payload_hbm.at[pl.ds(base, KCH)], rows_v)
        pltpu.async_copy(rows_v, grid_hbm.at[idx_v], sem).wait()


@functools.partial(
    pl.kernel,
    out_type=jax.ShapeDtypeStruct((NPAD, COUT), jnp.float32),
    mesh=_sc_mesh,
    scratch_types=[
        pltpu.VMEM((KCH,), jnp.int32),
        pltpu.VMEM((KCH, COUT), jnp.float32),
        pltpu.SemaphoreType.DMA,
    ],
)
def _sc_gather(idx_hbm, conv_hbm, out_hbm, idx_v, rows_v, sem):
    wid = lax.axis_index("s") * 2 + lax.axis_index("c")
    for j in range(PER_TILE // KCH):
        base = wid * PER_TILE + j * KCH
        pltpu.sync_copy(idx_hbm.at[pl.ds(base, KCH)], idx_v)
        pltpu.async_copy(conv_hbm.at[idx_v], rows_v, sem).wait()
        pltpu.sync_copy(rows_v, out_hbm.at[pl.ds(base, KCH)])


def _conv_body(pa, pb, ma, mb, w_ref, b_ref, out_ref, sm, smx):
    for p in range(BZ + 2):
        if p < BZ:
            src = pa[p * PLANE_S:p * PLANE_S + PLANE, :]
            m = ma[p, 0, :PLANE]
        else:
            src = pb[(p - BZ) * PLANE_S:(p - BZ) * PLANE_S + PLANE, :]
            m = mb[p - BZ, 0, :PLANE]
        keep = m.reshape(PLANE, 1) > 0.0
        sm[p] = jnp.where(keep, src, 0.0).astype(jnp.bfloat16).reshape(
            PD, PD, CROW)
        for dxi in range(3):
            smx[p, :, :, dxi * CROW:(dxi + 1) * CROW] = (
                sm[p, :, dxi:dxi + W, :])
    for oz in range(BZ):
        acc = jnp.broadcast_to(b_ref[0], (D * W, COUT)).astype(jnp.float32)
        for dzi in range(3):
            for dyi in range(3):
                g = dzi * 3 + dyi
                a = smx[oz + dzi, dyi:dyi + D, :, :].reshape(D * W, 3 * CROW)
                acc = acc + jnp.dot(a, w_ref[g],
                                    preferred_element_type=jnp.float32)
        out_ref[oz * D * W:(oz + 1) * D * W, :] = acc


def _conv(grid, mask, weight_p, bias):
    return pl.pallas_call(
        _conv_body,
        grid=(D // BZ,),
        in_specs=[
            pl.BlockSpec((BZ * PLANE_S, CROW), lambda z: (z, 0)),
            pl.BlockSpec((2 * PLANE_S, CROW),
                         lambda z: (BZ * (z + 1) // 2, 0)),
            pl.BlockSpec((BZ, 1, PLANE_S), lambda z: (z, 0, 0)),
            pl.BlockSpec((2, 1, PLANE_S),
                         lambda z: (BZ * (z + 1) // 2, 0, 0)),
            pl.BlockSpec((9, 3 * CROW, COUT), lambda z: (0, 0, 0)),
            pl.BlockSpec((1, COUT), lambda z: (0, 0)),
        ],
        out_specs=pl.BlockSpec((BZ * D * W, COUT), lambda z: (z, 0)),
        out_shape=jax.ShapeDtypeStruct((D * H * W, COUT), jnp.float32),
        scratch_shapes=[
            pltpu.VMEM((BZ + 2, PD, PD, CROW), jnp.bfloat16),
            pltpu.VMEM((BZ + 2, PD, W, 3 * CROW), jnp.bfloat16),
        ],
    )(grid, grid, mask, mask, weight_p, bias.reshape(1, COUT))


def _gates0_body(rows_ref, fnext_ref, h_ref, c_ref, pay_ref):
    rows = rows_ref[...]
    i = jax.nn.sigmoid(rows[:, 0 * CH:1 * CH])
    f = jax.nn.sigmoid(rows[:, 1 * CH:2 * CH])
    o = jax.nn.sigmoid(rows[:, 2 * CH:3 * CH])
    g = jnp.tanh(rows[:, 3 * CH:4 * CH])
    c = i * g
    del f
    h = o * jnp.tanh(c)
    h_ref[...] = h
    c_ref[...] = c
    pay_ref[:, 0:CIN] = fnext_ref[...]
    pay_ref[:, CIN:CIN + CH] = h
    pay_ref[:, CIN + CH:CROW] = jnp.zeros((_GBLK, CH), jnp.float32)


def _gates1_body(rows_ref, cprev_ref, h_ref, c_ref):
    rows = rows_ref[...]
    i = jax.nn.sigmoid(rows[:, 0 * CH:1 * CH])
    f = jax.nn.sigmoid(rows[:, 1 * CH:2 * CH])
    o = jax.nn.sigmoid(rows[:, 2 * CH:3 * CH])
    g = jnp.tanh(rows[:, 3 * CH:4 * CH])
    c = f * cprev_ref[...] + i * g
    h_ref[...] = h = o * jnp.tanh(c)
    c_ref[...] = c


_GBLK = 3136


def _gates0(rows, fnext):
    return pl.pallas_call(
        _gates0_body,
        grid=(NPAD // _GBLK,),
        in_specs=[
            pl.BlockSpec((_GBLK, COUT), lambda i: (i, 0)),
            pl.BlockSpec((_GBLK, CIN), lambda i: (i, 0)),
        ],
        out_specs=[
            pl.BlockSpec((_GBLK, CH), lambda i: (i, 0)),
            pl.BlockSpec((_GBLK, CH), lambda i: (i, 0)),
            pl.BlockSpec((_GBLK, CROW), lambda i: (i, 0)),
        ],
        out_shape=[
            jax.ShapeDtypeStruct((NPAD, CH), jnp.float32),
            jax.ShapeDtypeStruct((NPAD, CH), jnp.float32),
            jax.ShapeDtypeStruct((NPAD, CROW), jnp.float32),
        ],
    )(rows, fnext)


def _gates1(rows, cprev):
    return pl.pallas_call(
        _gates1_body,
        grid=(NPAD // _GBLK,),
        in_specs=[
            pl.BlockSpec((_GBLK, COUT), lambda i: (i, 0)),
            pl.BlockSpec((_GBLK, CH), lambda i: (i, 0)),
        ],
        out_specs=[
            pl.BlockSpec((_GBLK, CH), lambda i: (i, 0)),
            pl.BlockSpec((_GBLK, CH), lambda i: (i, 0)),
        ],
        out_shape=[
            jax.ShapeDtypeStruct((NPAD, CH), jnp.float32),
            jax.ShapeDtypeStruct((NPAD, CH), jnp.float32),
        ],
    )(rows, cprev)


def kernel(features, coords, weight, bias):
    N = features.shape[1]
    z, y, x = coords[:, 0], coords[:, 1], coords[:, 2]
    lin64 = (z * H + y) * W + x
    plin = (z + 1) * PLANE_S + (y + 1) * PD + (x + 1)

    idxmap = jnp.full((D * H * W,), -1, jnp.int32).at[lin64].set(
        jnp.arange(N, dtype=jnp.int32))
    winner = idxmap[lin64] == jnp.arange(N, dtype=jnp.int32)
    iarange = jnp.arange(NPAD, dtype=jnp.int32)
    winner_pad = jnp.zeros((NPAD,), jnp.bool_).at[:N].set(winner)
    plin_full = jnp.zeros((NPAD,), jnp.int32).at[:N].set(plin)
    plin_pad = jnp.where(winner_pad, plin_full, NPS + iarange)
    lin_pad = jnp.zeros((NPAD,), jnp.int32).at[:N].set(lin64)

    mask = jnp.zeros((NPS,), jnp.float32).at[plin].set(1.0).reshape(
        PD, 1, PLANE_S)

    weight_p = jnp.zeros((27, CROW, COUT), jnp.bfloat16)
    weight_p = weight_p.at[:, :CIN + CH, :].set(weight.astype(jnp.bfloat16))
    weight_p = weight_p.reshape(9, 3 * CROW, COUT)

    fpad = jnp.zeros((2, NPAD, CIN), jnp.float32).at[:, :N, :].set(features)
    pay0 = jnp.pad(fpad[0], ((0, 0), (0, CROW - CIN)))

    grid0 = _sc_scatter(plin_pad, pay0)
    conv0 = _conv(grid0, mask, weight_p, bias)
    rows0 = _sc_gather(lin_pad, conv0)
    h0, c0, pay1 = _gates0(rows0, fpad[1])

    grid1 = _sc_scatter(plin_pad, pay1)
    conv1 = _conv(grid1, mask, weight_p, bias)
    rows1 = _sc_gather(lin_pad, conv1)
    h1, c1 = _gates1(rows1, c0)

    outs = jnp.stack([h0[:N], h1[:N]])
    return (outs, h1[:N], c1[:N])
